# baseline (speedup 1.0000x reference)
import functools

import jax
import jax.numpy as jnp
from jax import lax
from jax.experimental import pallas as pl
from jax.experimental.pallas import tpu as pltpu
from jax.experimental.pallas import tpu_sc as plsc

N_NODES = 10000
N_EDGES = 320000
IN_DIM = 128
HID = 16

NC = 2
NS = 16
NW = NC * NS
E_PER_TILE = N_EDGES // NW
N_PAD = 10240
ROWS_PER_TILE = N_PAD // NS


PACK = 8
NPROW = N_PAD // PACK
_f32 = jnp.float32
_packed = jax.ShapeDtypeStruct((NPROW, PACK * HID), _f32)


def _tc1_body(xp_ref, wbd_ref, b_ref, y1_ref, r1_ref):
    y = jnp.dot(xp_ref[...], wbd_ref[...], preferred_element_type=jnp.float32)
    y1_ref[...] = y[:, :PACK * HID]
    r1_ref[...] = y[:, PACK * HID:] + jnp.concatenate([b_ref[...]] * PACK, axis=1)


def _tc2_body(p_ref, r1_ref, wbd_ref, b_ref, y2_ref, r2_ref):
    h = jnp.maximum(p_ref[0] + p_ref[1] + r1_ref[...], 0.0)
    y = jnp.dot(h, wbd_ref[...], preferred_element_type=jnp.float32)
    y2_ref[...] = y[:, :PACK * HID]
    r2_ref[...] = y[:, PACK * HID:] + jnp.concatenate([b_ref[...]] * PACK, axis=1)


def _tc3_body(p_ref, r2_ref, out_ref):
    out_ref[...] = jnp.maximum(p_ref[0, :N_NODES // PACK]
                               + p_ref[1, :N_NODES // PACK]
                               + r2_ref[:N_NODES // PACK], 0.0)


_TCG = 8
_BR = NPROW // _TCG

_tc1 = pl.pallas_call(
    _tc1_body,
    grid=(_TCG,),
    in_specs=[pl.BlockSpec((_BR, PACK * IN_DIM), lambda i: (i, 0)),
              pl.BlockSpec((PACK * IN_DIM, 2 * PACK * HID), lambda i: (0, 0)),
              pl.BlockSpec((1, HID), lambda i: (0, 0))],
    out_specs=(pl.BlockSpec((_BR, PACK * HID), lambda i: (i, 0)),
               pl.BlockSpec((_BR, PACK * HID), lambda i: (i, 0))),
    out_shape=(_packed, _packed))
_tc2 = pl.pallas_call(
    _tc2_body,
    grid=(_TCG,),
    in_specs=[pl.BlockSpec((NC, _BR, PACK * HID), lambda i: (0, i, 0)),
              pl.BlockSpec((_BR, PACK * HID), lambda i: (i, 0)),
              pl.BlockSpec((PACK * HID, 2 * PACK * HID), lambda i: (0, 0)),
              pl.BlockSpec((1, HID), lambda i: (0, 0))],
    out_specs=(pl.BlockSpec((_BR, PACK * HID), lambda i: (i, 0)),
               pl.BlockSpec((_BR, PACK * HID), lambda i: (i, 0))),
    out_shape=(_packed, _packed))
_tc3 = pl.pallas_call(
    _tc3_body, out_shape=jax.ShapeDtypeStruct((N_NODES // PACK, PACK * HID), _f32))


SUP = 1000
NSUP = E_PER_TILE // SUP
NBANK = 4


def _sc_body(y_hbm, edges_hbm, zeros_hbm, out_hbm,
             src_v, dst_v, banks, sgs, sss, acc_sh, y_sh):
    cid = lax.axis_index("c")
    sid = lax.axis_index("s")
    wid = sid * NC + cid

    rbase = sid * ROWS_PER_TILE
    pltpu.sync_copy(zeros_hbm.at[pl.ds(rbase, ROWS_PER_TILE)],
                    acc_sh.at[pl.ds(rbase, ROWS_PER_TILE)])

    pltpu.sync_copy(y_hbm.at[pl.ds(rbase, ROWS_PER_TILE)],
                    y_sh.at[pl.ds(rbase, ROWS_PER_TILE)])

    ebase = wid * E_PER_TILE
    pltpu.sync_copy(edges_hbm.at[pl.ds(ebase, E_PER_TILE)], src_v)
    pltpu.sync_copy(edges_hbm.at[pl.ds(N_EDGES + ebase, E_PER_TILE)], dst_v)
    plsc.subcore_barrier()

    def fire_gather(g):
        b = g % NBANK
        return pltpu.async_copy(
            y_sh.at[src_v.at[pl.ds(g * SUP, SUP)]], banks[b], sgs[b])

    def fire_scatter(g):
        b = g % NBANK
        return pltpu.async_copy(
            banks[b], acc_sh.at[dst_v.at[pl.ds(g * SUP, SUP)]], sss[b],
            add=True)

    AHEAD = NBANK - 1
    gd = {g: fire_gather(g) for g in range(min(AHEAD, NSUP))}
    sd = {}
    for g in range(NSUP):
        gd[g].wait()
        sd[g] = fire_scatter(g)
        if g + AHEAD < NSUP:
            if g + AHEAD - NBANK >= 0:
                sd[g + AHEAD - NBANK].wait()
            gd[g + AHEAD] = fire_gather(g + AHEAD)
    for g in range(max(0, NSUP - NBANK), NSUP):
        sd[g].wait()

    plsc.subcore_barrier()
    pltpu.sync_copy(acc_sh.at[pl.ds(rbase, ROWS_PER_TILE)],
                    out_hbm.at[cid, pl.ds(rbase, ROWS_PER_TILE)])


_sc_seg_sum = functools.partial(
    pl.kernel,
    out_type=jax.ShapeDtypeStruct((NC, N_PAD, HID), _f32),
    mesh=plsc.VectorSubcoreMesh(core_axis_name="c", subcore_axis_name="s",
                                num_cores=NC, num_subcores=NS),
    compiler_params=pltpu.CompilerParams(use_tc_tiling_on_sc=False,
                                         disable_bounds_checks=True),
    scratch_types=[
        pltpu.VMEM((E_PER_TILE,), jnp.int32),
        pltpu.VMEM((E_PER_TILE,), jnp.int32),
        [pltpu.VMEM((SUP, HID), _f32) for _ in range(NBANK)],
        [pltpu.SemaphoreType.DMA for _ in range(NBANK)],
        [pltpu.SemaphoreType.DMA for _ in range(NBANK)],
        pltpu.VMEM_SHARED((N_PAD, HID), _f32),
        pltpu.VMEM_SHARED((N_PAD, HID), _f32),
    ],
)(_sc_body)


def kernel(x, edge_index, W1_rel, W1_root, b1, W2_rel, W2_root, b2):
    edges = edge_index.astype(jnp.int32).reshape(2 * N_EDGES)
    zeros = jnp.zeros((N_PAD, HID), _f32)

    eye8 = jnp.eye(PACK, dtype=_f32)
    w1bd = jnp.concatenate([jnp.kron(eye8, W1_rel), jnp.kron(eye8, W1_root)],
                           axis=1)
    w2bd = jnp.concatenate([jnp.kron(eye8, W2_rel), jnp.kron(eye8, W2_root)],
                           axis=1)
    b1r = b1.reshape(1, HID)
    b2r = b2.reshape(1, HID)

    xp = jnp.pad(x, ((0, N_PAD - N_NODES), (0, 0))).reshape(NPROW, PACK * IN_DIM)
    y1p, r1p = _tc1(xp, w1bd, b1r)
    p = _sc_seg_sum(y1p.reshape(N_PAD, HID), edges, zeros)
    y2p, r2p = _tc2(p.reshape(NC, NPROW, PACK * HID), r1p, w2bd, b2r)
    p2 = _sc_seg_sum(y2p.reshape(N_PAD, HID), edges, zeros)
    outp = _tc3(p2.reshape(NC, NPROW, PACK * HID), r2p)
    return outp.reshape(N_NODES, HID)


# speedup vs baseline: 1.0707x; 1.0707x over previous
import functools

import jax
import jax.numpy as jnp
from jax import lax
from jax.experimental import pallas as pl
from jax.experimental.pallas import tpu as pltpu
from jax.experimental.pallas import tpu_sc as plsc

N_NODES = 10000
N_EDGES = 320000
IN_DIM = 128
HID = 16

NC = 2
NS = 16
NW = NC * NS
E_PER_TILE = N_EDGES // NW
N_PAD = 10240
ROWS_PER_TILE = N_PAD // NS


PACK = 8
NPROW = N_PAD // PACK
_f32 = jnp.float32
_packed = jax.ShapeDtypeStruct((NPROW, PACK * HID), _f32)


def _tc1_body(xp_ref, wbd_ref, b_ref, y1_ref, r1_ref):
    y = jnp.dot(xp_ref[...], wbd_ref[...], preferred_element_type=jnp.float32)
    y1_ref[...] = y[:, :PACK * HID]
    r1_ref[...] = y[:, PACK * HID:] + jnp.concatenate([b_ref[...]] * PACK, axis=1)


def _tc2_body(p_ref, r1_ref, wbd_ref, b_ref, y2_ref, r2_ref):
    h = jnp.maximum(p_ref[0] + p_ref[1] + r1_ref[...], 0.0)
    y = jnp.dot(h, wbd_ref[...], preferred_element_type=jnp.float32)
    y2_ref[...] = y[:, :PACK * HID]
    r2_ref[...] = y[:, PACK * HID:] + jnp.concatenate([b_ref[...]] * PACK, axis=1)


def _tc3_body(p_ref, r2_ref, out_ref):
    out_ref[...] = jnp.maximum(p_ref[0, :N_NODES // PACK]
                               + p_ref[1, :N_NODES // PACK]
                               + r2_ref[:N_NODES // PACK], 0.0)


_tc1 = pl.pallas_call(_tc1_body, out_shape=(_packed, _packed))
_tc2 = pl.pallas_call(_tc2_body, out_shape=(_packed, _packed))
_tc3 = pl.pallas_call(
    _tc3_body, out_shape=jax.ShapeDtypeStruct((N_NODES // PACK, PACK * HID), _f32))


SUP = 1000
NSUP = E_PER_TILE // SUP
NBANK = 4


def _sc_body(y_hbm, edges_hbm, zeros_hbm, out_hbm,
             src_v, dst_v, banks, sgs, sss, acc_sh, y_sh):
    cid = lax.axis_index("c")
    sid = lax.axis_index("s")
    wid = sid * NC + cid

    rbase = sid * ROWS_PER_TILE
    pltpu.sync_copy(zeros_hbm.at[pl.ds(rbase, ROWS_PER_TILE)],
                    acc_sh.at[pl.ds(rbase, ROWS_PER_TILE)])

    pltpu.sync_copy(y_hbm.at[pl.ds(rbase, ROWS_PER_TILE)],
                    y_sh.at[pl.ds(rbase, ROWS_PER_TILE)])

    ebase = wid * E_PER_TILE
    pltpu.sync_copy(edges_hbm.at[pl.ds(ebase, E_PER_TILE)], src_v)
    pltpu.sync_copy(edges_hbm.at[pl.ds(N_EDGES + ebase, E_PER_TILE)], dst_v)
    plsc.subcore_barrier()

    def fire_gather(g):
        b = g % NBANK
        return pltpu.async_copy(
            y_sh.at[src_v.at[pl.ds(g * SUP, SUP)]], banks[b], sgs[b])

    def fire_scatter(g):
        b = g % NBANK
        return pltpu.async_copy(
            banks[b], acc_sh.at[dst_v.at[pl.ds(g * SUP, SUP)]], sss[b],
            add=True)

    AHEAD = NBANK - 1
    gd = {g: fire_gather(g) for g in range(min(AHEAD, NSUP))}
    sd = {}
    for g in range(NSUP):
        gd[g].wait()
        sd[g] = fire_scatter(g)
        if g + AHEAD < NSUP:
            if g + AHEAD - NBANK >= 0:
                sd[g + AHEAD - NBANK].wait()
            gd[g + AHEAD] = fire_gather(g + AHEAD)
    for g in range(max(0, NSUP - NBANK), NSUP):
        sd[g].wait()

    plsc.subcore_barrier()
    pltpu.sync_copy(acc_sh.at[pl.ds(rbase, ROWS_PER_TILE)],
                    out_hbm.at[cid, pl.ds(rbase, ROWS_PER_TILE)])


_sc_seg_sum = functools.partial(
    pl.kernel,
    out_type=jax.ShapeDtypeStruct((NC, N_PAD, HID), _f32),
    mesh=plsc.VectorSubcoreMesh(core_axis_name="c", subcore_axis_name="s",
                                num_cores=NC, num_subcores=NS),
    compiler_params=pltpu.CompilerParams(use_tc_tiling_on_sc=False,
                                         disable_bounds_checks=True),
    scratch_types=[
        pltpu.VMEM((E_PER_TILE,), jnp.int32),
        pltpu.VMEM((E_PER_TILE,), jnp.int32),
        [pltpu.VMEM((SUP, HID), _f32) for _ in range(NBANK)],
        [pltpu.SemaphoreType.DMA for _ in range(NBANK)],
        [pltpu.SemaphoreType.DMA for _ in range(NBANK)],
        pltpu.VMEM_SHARED((N_PAD, HID), _f32),
        pltpu.VMEM_SHARED((N_PAD, HID), _f32),
    ],
)(_sc_body)


def kernel(x, edge_index, W1_rel, W1_root, b1, W2_rel, W2_root, b2):
    edges = edge_index.astype(jnp.int32).reshape(2 * N_EDGES)
    zeros = jnp.zeros((N_PAD, HID), _f32)

    eye8 = jnp.eye(PACK, dtype=_f32)
    w1bd = jnp.concatenate([jnp.kron(eye8, W1_rel), jnp.kron(eye8, W1_root)],
                           axis=1)
    w2bd = jnp.concatenate([jnp.kron(eye8, W2_rel), jnp.kron(eye8, W2_root)],
                           axis=1)
    b1r = b1.reshape(1, HID)
    b2r = b2.reshape(1, HID)

    xp = jnp.pad(x, ((0, N_PAD - N_NODES), (0, 0))).reshape(NPROW, PACK * IN_DIM)
    y1p, r1p = _tc1(xp, w1bd, b1r)
    p = _sc_seg_sum(y1p.reshape(N_PAD, HID), edges, zeros)
    y2p, r2p = _tc2(p.reshape(NC, NPROW, PACK * HID), r1p, w2bd, b2r)
    p2 = _sc_seg_sum(y2p.reshape(N_PAD, HID), edges, zeros)
    outp = _tc3(p2.reshape(NC, NPROW, PACK * HID), r2p)
    return outp.reshape(N_NODES, HID)
